# baseline (speedup 1.0000x reference)
import jax
import jax.numpy as jnp
from jax import lax
from jax.experimental import pallas as pl
from jax.experimental.pallas import tpu as pltpu
from jax.experimental.pallas import tpu_sc as plsc

_DIM = 64
_LANES = 128
_NC = 2
_NS = 16
_NW = _NC * _NS


def _gather_body(tbl_hbm, idx_hbm, out_hbm,
                 idx_slab, glist, par, gbuf, obuf,
                 gsem0, gsem1, wsem0, wsem1):
    L = out_hbm.shape[0]
    BT = out_hbm.shape[2]
    bcols = BT // _NW
    mt_per_w = bcols // _LANES
    n_steps = L * mt_per_w
    slab = bcols * L

    wid = lax.axis_index("s") * _NC + lax.axis_index("c")
    b0 = wid * bcols

    gsems = (gsem0, gsem1)
    wsems = (wsem0, wsem1)

    pltpu.sync_copy(idx_hbm.at[pl.ds(wid * slab, slab)], idx_slab)

    iota = lax.iota(jnp.int32, 16)

    def build_lists(t, b):
        l = t >> 2
        mm = t & (mt_per_w - 1)
        for g in range(8):
            offs = (mm * _LANES + g * 16 + iota) * L + l
            v = plsc.load_gather(idx_slab, [offs])
            row = lax.shift_right_logical(v, 1)
            parv = lax.shift_left(lax.bitwise_and(v, 1), 6)
            glist.at[b][pl.ds(g * 16, 16)] = row
            par.at[b][pl.ds(g * 16, 16)] = parv

    def start_gather(b):
        return pltpu.async_copy(tbl_hbm.at[glist.at[b]], gbuf.at[b], gsems[b])

    def transpose(b):
        par_ref = par.at[b]
        gb = gbuf.at[b]
        ob = obuf.at[b]
        rowvs = [g * 16 + iota for g in range(8)]
        parvs = [plsc.load_gather(par_ref, [rowvs[g]]) for g in range(8)]

        @plsc.parallel_loop(0, _DIM, unroll=8)
        def _(c):
            for g in range(8):
                val = plsc.load_gather(gb, [rowvs[g], parvs[g] + c])
                ob[c, pl.ds(g * 16, 16)] = val

    def start_write(t, b):
        l = t >> 2
        mm = t & (mt_per_w - 1)
        return pltpu.async_copy(
            obuf.at[b],
            out_hbm.at[l, :, pl.ds(b0 + mm * _LANES, _LANES)],
            wsems[b],
        )

    def wait_write(b):
        pltpu.make_async_copy(
            obuf.at[b], out_hbm.at[0, :, pl.ds(b0, _LANES)], wsems[b]
        ).wait()

    @pl.loop(0, n_steps // 2)
    def _(s):
        descs = []
        for b in range(2):
            t = 2 * s + b

            @pl.when(s > 0)
            def _():
                wait_write(b)

            build_lists(t, b)
            descs.append(start_gather(b))
        for b in range(2):
            t = 2 * s + b
            descs[b].wait()
            transpose(b)
            start_write(t, b)

    for b in range(2):
        wait_write(b)


def kernel(input, table):
    B, L = input.shape
    V = table.shape[0]
    n = B * L
    idx = input.reshape(n).astype(jnp.int32)
    tbl2 = table.reshape(V // 2, 2 * _DIM)
    mesh = plsc.VectorSubcoreMesh(core_axis_name="c", subcore_axis_name="s")
    gather = pl.kernel(
        _gather_body,
        out_type=jax.ShapeDtypeStruct((L, _DIM, B), jnp.float32),
        mesh=mesh,
        scratch_types=[
            pltpu.VMEM((n // _NW,), jnp.int32),
            pltpu.VMEM((2, _LANES), jnp.int32),
            pltpu.VMEM((2, _LANES), jnp.int32),
            pltpu.VMEM((2, _LANES, _LANES), jnp.float32),
            pltpu.VMEM((2, _DIM, _LANES), jnp.float32),
            pltpu.SemaphoreType.DMA,
            pltpu.SemaphoreType.DMA,
            pltpu.SemaphoreType.DMA,
            pltpu.SemaphoreType.DMA,
        ],
        compiler_params=pltpu.CompilerParams(needs_layout_passes=False),
    )
    out_t = gather(tbl2, idx)
    return out_t.transpose(2, 0, 1)
